# baseline (speedup 1.0000x reference)
import functools

import jax
import jax.numpy as jnp
from jax import lax
from jax.experimental import pallas as pl
from jax.experimental.pallas import tpu as pltpu, tpu_sc as plsc

_D_MODEL = 4096
_NUM_PARTITIONS = 64
_K = 8
_TOKEN_BLOCK = 1024
_N_TOK = 16384
_N_WORKERS = 32
_TOK_PER_W = _N_TOK // _N_WORKERS
_L = 16


def _score_t_kernel(x_ref, wt_ref, b_ref, out_ref):
    scores = jnp.dot(x_ref[...], wt_ref[...],
                     preferred_element_type=jnp.float32)
    scores = scores + b_ref[...]
    out_ref[...] = scores.T


def _tc_scores_t(x2, wt, b2):
    grid = (_N_TOK // _TOKEN_BLOCK,)
    return pl.pallas_call(
        _score_t_kernel,
        grid=grid,
        in_specs=[
            pl.BlockSpec((_TOKEN_BLOCK, _D_MODEL), lambda i: (i, 0)),
            pl.BlockSpec((_D_MODEL, _NUM_PARTITIONS), lambda i: (0, 0)),
            pl.BlockSpec((1, _NUM_PARTITIONS), lambda i: (0, 0)),
        ],
        out_specs=pl.BlockSpec((_NUM_PARTITIONS, _TOKEN_BLOCK),
                               lambda i: (0, i)),
        out_shape=jax.ShapeDtypeStruct((_NUM_PARTITIONS, _N_TOK), jnp.float32),
    )(x2, wt, b2)


_SC_MESH = plsc.VectorSubcoreMesh(core_axis_name="c", subcore_axis_name="s")


@functools.partial(
    pl.kernel,
    mesh=_SC_MESH,
    out_type=jax.ShapeDtypeStruct((_K, _N_TOK), jnp.int32),
    scratch_types=[
        pltpu.VMEM((_NUM_PARTITIONS * _TOK_PER_W,), jnp.float32),
        pltpu.VMEM((_K * _TOK_PER_W,), jnp.int32),
    ],
)
def _sc_topk(st_hbm, out_hbm, buf, out_buf):
    wid = lax.axis_index("s") * 2 + lax.axis_index("c")
    base = wid * _TOK_PER_W
    for p in range(_NUM_PARTITIONS):
        pltpu.sync_copy(st_hbm.at[p, pl.ds(base, _TOK_PER_W)],
                        buf.at[pl.ds(p * _TOK_PER_W, _TOK_PER_W)])

    neg_inf = jnp.full((_L,), -jnp.inf, jnp.float32)
    zero_i = jnp.zeros((_L,), jnp.int32)

    def group_body(g, carry):
        goff = g * _L
        vals = [neg_inf] * _K
        inds = [zero_i] * _K
        for p in range(_NUM_PARTITIONS):
            v = buf[pl.ds(p * _TOK_PER_W + goff, _L)]
            vi = jnp.full((_L,), p, jnp.int32)
            for j in range(_K):
                gt = v > vals[j]
                nv = jnp.where(gt, v, vals[j])
                v = jnp.where(gt, vals[j], v)
                vals[j] = nv
                ni = jnp.where(gt, vi, inds[j])
                vi = jnp.where(gt, inds[j], vi)
                inds[j] = ni
        for j in range(_K):
            out_buf[pl.ds(j * _TOK_PER_W + goff, _L)] = inds[j]
        return carry

    lax.fori_loop(0, _TOK_PER_W // _L, group_body, 0)
    for j in range(_K):
        pltpu.sync_copy(out_buf.at[pl.ds(j * _TOK_PER_W, _TOK_PER_W)],
                        out_hbm.at[j, pl.ds(base, _TOK_PER_W)])


def kernel(x, W, b):
    bsz, seq, d = x.shape
    x2 = x.reshape(bsz * seq, d)
    wt = W.T
    b2 = b.reshape(1, _NUM_PARTITIONS)
    st = _tc_scores_t(x2, wt, b2)
    out = _sc_topk(st)
    return out.T.reshape(bsz, seq, _K)


# speedup vs baseline: 1.2860x; 1.2860x over previous
import functools

import jax
import jax.numpy as jnp
from jax.experimental import pallas as pl
from jax.experimental.pallas import tpu as pltpu

_D_MODEL = 4096
_NUM_PARTITIONS = 64
_K = 8
_TOKEN_BLOCK = 1024
_D_CHUNKS = 2
_D_CHUNK = _D_MODEL // _D_CHUNKS


def _router_topk_kernel(x_ref, wt_ref, b_ref, out_ref, acc_ref):
    j = pl.program_id(1)
    part = jnp.dot(x_ref[...], wt_ref[...],
                   preferred_element_type=jnp.float32).T

    @pl.when(j == 0)
    def _init():
        acc_ref[...] = part

    @pl.when(j > 0)
    def _acc():
        acc_ref[...] += part

    @pl.when(j == _D_CHUNKS - 1)
    def _epilogue():
        st = acc_ref[...] + b_ref[...]
        row = jax.lax.broadcasted_iota(jnp.int32, st.shape, 0)
        neg_inf = jnp.float32(-jnp.inf)
        big = jnp.int32(_NUM_PARTITIONS)
        idxs = []
        for _ in range(_K):
            m = jnp.max(st, axis=0)
            eq = st == m[None, :]
            idx = jnp.min(jnp.where(eq, row, big), axis=0)
            idxs.append(idx)
            st = jnp.where(row == idx[None, :], neg_inf, st)
        out_ref[...] = jnp.stack(idxs, axis=1)


@functools.partial(jax.jit, static_argnames=())
def kernel(x, W, b):
    bsz, seq, d = x.shape
    n_tok = bsz * seq
    x2 = x.reshape(n_tok, d)
    wt = W.T
    b2 = b.reshape(_NUM_PARTITIONS, 1)
    grid = (n_tok // _TOKEN_BLOCK, _D_CHUNKS)
    out = pl.pallas_call(
        _router_topk_kernel,
        grid=grid,
        in_specs=[
            pl.BlockSpec((_TOKEN_BLOCK, _D_CHUNK), lambda i, j: (i, j)),
            pl.BlockSpec((_D_CHUNK, _NUM_PARTITIONS), lambda i, j: (j, 0)),
            pl.BlockSpec((_NUM_PARTITIONS, 1), lambda i, j: (0, 0)),
        ],
        out_specs=pl.BlockSpec((_TOKEN_BLOCK, _K), lambda i, j: (i, 0)),
        out_shape=jax.ShapeDtypeStruct((n_tok, _K), jnp.int32),
        scratch_shapes=[pltpu.VMEM((_NUM_PARTITIONS, _TOKEN_BLOCK),
                                   jnp.float32)],
    )(x2, wt, b2)
    return out.reshape(bsz, seq, _K)


# speedup vs baseline: 1.5998x; 1.2440x over previous
import functools

import jax
import jax.numpy as jnp
from jax.experimental import pallas as pl

_D_MODEL = 4096
_NUM_PARTITIONS = 64
_K = 8
_TOKEN_BLOCK = 1024


def _router_topk_kernel(x_ref, wt_ref, b_ref, out_ref):
    scores = jnp.dot(x_ref[...], wt_ref[...],
                     preferred_element_type=jnp.float32)
    scores = scores + b_ref[...]
    st = scores.T
    row = jax.lax.broadcasted_iota(jnp.int32, st.shape, 0)
    neg_inf = jnp.float32(-jnp.inf)
    big = jnp.int32(_NUM_PARTITIONS)
    idxs = []
    for _ in range(_K):
        m = jnp.max(st, axis=0)
        eq = st == m[None, :]
        idx = jnp.min(jnp.where(eq, row, big), axis=0)
        idxs.append(idx)
        st = jnp.where(row == idx[None, :], neg_inf, st)
    out_ref[...] = jnp.stack(idxs, axis=1)


@functools.partial(jax.jit, static_argnames=())
def kernel(x, W, b):
    bsz, seq, d = x.shape
    n_tok = bsz * seq
    x2 = x.reshape(n_tok, d)
    wt = W.T
    b2 = b.reshape(1, _NUM_PARTITIONS)
    grid = (n_tok // _TOKEN_BLOCK,)
    out = pl.pallas_call(
        _router_topk_kernel,
        grid=grid,
        in_specs=[
            pl.BlockSpec((_TOKEN_BLOCK, d), lambda i: (i, 0)),
            pl.BlockSpec((d, _NUM_PARTITIONS), lambda i: (0, 0)),
            pl.BlockSpec((1, _NUM_PARTITIONS), lambda i: (0, 0)),
        ],
        out_specs=pl.BlockSpec((_TOKEN_BLOCK, _K), lambda i: (i, 0)),
        out_shape=jax.ShapeDtypeStruct((n_tok, _K), jnp.int32),
    )(x2, wt, b2)
    return out.reshape(bsz, seq, _K)
